# baseline (speedup 1.0000x reference)
import jax
import jax.numpy as jnp
from jax import lax
from jax.experimental import pallas as pl
from jax.experimental.pallas import tpu as pltpu
from jax.experimental.pallas import tpu_sc as plsc

N_USER = 50000
N_ITEM = 10000
D = 128
E = 320000

NC = 2
NS = 16
EP = E // NS
BLK_E = 2000
NBLK = EP // BLK_E
CHUNK = 96
NSLOT = 3
CCAP = 2208
CTRASH = CCAP - 16
SP_ROWS = 8448
ZPT = 528
TRASH_ROW = 8336

U_SZ_MAIN = 8336
U_SZ_LAST = 8320
IR = 5000


def _sc_body(x_user, x_item, src_u2i, dst_u2i, w_u2i, src_i2u, dst_i2u,
             w_i2u, agg_item, agg_user, bdst, bsrc, bw, cdst, csrc, cw,
             didx, sidx, rows, spacc, gsem, ssem):
    c = lax.axis_index("c")
    s = lax.axis_index("s")

    def do_pass(table, src_hbm, dst_hbm, w_hbm, lo, hi, flush):
        def stage_fire(i, nb):
            base = i * CHUNK
            for j in range(CHUNK // 16):
                didx[nb, pl.ds(j * 16, 16)] = cdst[pl.ds(base + j * 16, 16)]
                sidx[nb, pl.ds(j * 16, 16)] = csrc[pl.ds(base + j * 16, 16)]
            pltpu.async_copy(table.at[sidx.at[nb]], rows.at[nb],
                             gsem.at[nb])

        def scale_scatter(i, b):
            base = i * CHUNK

            def scale(e, _):
                w = plsc.load_gather(cw,
                                     [jnp.full((16,), base + e, jnp.int32)])
                for j in range(8):
                    sl = pl.ds(j * 16, 16)
                    rows[b, e, sl] = rows[b, e, sl] * w
                return 0
            lax.fori_loop(0, CHUNK, scale, 0, unroll=4)
            pltpu.sync_copy(rows.at[b], spacc.at[didx.at[b]], add=True)

        def run_chunks(n_full):
            @pl.when(n_full > 0)
            def _():
                stage_fire(0, 0)

            @pl.when(n_full > 1)
            def _():
                stage_fire(1, 1)

            def ch_body(i, _):
                b = lax.rem(i, NSLOT)
                pltpu.make_async_copy(table.at[sidx.at[b]], rows.at[b],
                                      gsem.at[b]).wait()

                @pl.when(i + 2 < n_full)
                def _():
                    stage_fire(i + 2, lax.rem(i + 2, NSLOT))
                scale_scatter(i, b)
                return 0
            lax.fori_loop(0, n_full, ch_body, 0)

        with jax.named_scope("zero"):
            def _zrow(r, _):
                for j in range(8):
                    rows[0, r, pl.ds(j * 16, 16)] = jnp.zeros((16,),
                                                              jnp.float32)
                return 0
            lax.fori_loop(0, CHUNK, _zrow, 0)
            zbase = s * ZPT
            for k in range(5):
                pltpu.sync_copy(rows.at[0],
                                spacc.at[pl.ds(zbase + k * 96, 96)])
            pltpu.sync_copy(rows.at[0].at[pl.ds(0, 48)],
                            spacc.at[pl.ds(zbase + 480, 48)])
        with jax.named_scope("barrier0"):
            plsc.subcore_barrier()
        ep0 = s * EP
        pltpu.async_copy(dst_hbm.at[pl.ds(ep0, BLK_E)], bdst, ssem)
        pltpu.async_copy(src_hbm.at[pl.ds(ep0, BLK_E)], bsrc, ssem)
        pltpu.async_copy(w_hbm.at[pl.ds(ep0, BLK_E)], bw, ssem)

        def vec_body(v, cnt):
            sl = pl.ds(v * 16, 16)
            d = bdst[sl]
            m = (d >= lo) & (d < hi)
            ms = m.astype(jnp.int32)
            pref = lax.cumsum(ms)
            pos = jnp.where(m, cnt + pref - ms,
                            CTRASH + lax.iota(jnp.int32, 16))
            plsc.store_scatter(cdst, [pos], d - lo)
            plsc.store_scatter(csrc, [pos], bsrc[sl])
            plsc.store_scatter(cw, [pos], bw[sl])
            return cnt + plsc.all_reduce_population_count(m)

        def blk_body(blk, cnt):
            off = s * EP + blk * BLK_E
            with jax.named_scope("stream"):
                pltpu.make_async_copy(dst_hbm.at[pl.ds(off, BLK_E)], bdst,
                                      ssem).wait()
                pltpu.make_async_copy(src_hbm.at[pl.ds(off, BLK_E)], bsrc,
                                      ssem).wait()
                pltpu.make_async_copy(w_hbm.at[pl.ds(off, BLK_E)], bw,
                                      ssem).wait()
            with jax.named_scope("scan"):
                cnt = lax.fori_loop(0, BLK_E // 16, vec_body, cnt, unroll=2)

            @pl.when(blk + 1 < NBLK)
            def _():
                off2 = off + BLK_E
                pltpu.async_copy(dst_hbm.at[pl.ds(off2, BLK_E)], bdst, ssem)
                pltpu.async_copy(src_hbm.at[pl.ds(off2, BLK_E)], bsrc, ssem)
                pltpu.async_copy(w_hbm.at[pl.ds(off2, BLK_E)], bw, ssem)
            n_full = cnt[0] // CHUNK
            with jax.named_scope("chunks"):
                run_chunks(n_full)
            with jax.named_scope("rem"):
                rbase = n_full * CHUNK
                for j in range(CHUNK // 16):
                    so = pl.ds(rbase + j * 16, 16)
                    do = pl.ds(j * 16, 16)
                    dv, sv, wv = cdst[so], csrc[so], cw[so]
                    cdst[do] = dv
                    csrc[do] = sv
                    cw[do] = wv
            return cnt - rbase

        cntv = lax.fori_loop(0, NBLK, blk_body,
                             jnp.zeros((16,), jnp.int32))
        cnt = cntv[0]
        with jax.named_scope("tail"):
            trash = jnp.full((16,), TRASH_ROW + s, jnp.int32)
            zf = jnp.zeros((16,), jnp.float32)
            for i in range(CHUNK // 16):
                off = pl.ds(cnt + i * 16, 16)
                cdst[off] = trash
                csrc[off] = (s * CHUNK + i * 16) + lax.iota(jnp.int32, 16)
                cw[off] = zf
            run_chunks((cnt + CHUNK - 1) // CHUNK)
        with jax.named_scope("barrier1"):
            plsc.subcore_barrier()
        with jax.named_scope("flush"):
            flush()
        with jax.named_scope("barrier2"):
            plsc.subcore_barrier()

    ilo = c * IR

    def flush_item():
        @pl.when(s < 15)
        def _():
            pltpu.sync_copy(spacc.at[pl.ds(s * 312, 312)],
                            agg_item.at[pl.ds(ilo + s * 312, 312)])

        @pl.when(s == 15)
        def _():
            pltpu.sync_copy(spacc.at[pl.ds(15 * 312, 320)],
                            agg_item.at[pl.ds(ilo + 15 * 312, 320)])

    do_pass(x_user, src_u2i, dst_u2i, w_u2i, ilo, ilo + IR, flush_item)

    for p in range(3):
        rid = c * 3 + p
        ulo = rid * U_SZ_MAIN
        usz = jnp.where(rid == 5, U_SZ_LAST, U_SZ_MAIN).astype(jnp.int32)

        def flush_user(ulo=ulo, rid=rid):
            @pl.when(rid < 5)
            def _():
                @pl.when(s < 15)
                def _():
                    pltpu.sync_copy(spacc.at[pl.ds(s * 528, 528)],
                                    agg_user.at[pl.ds(ulo + s * 528, 528)])

                @pl.when(s == 15)
                def _():
                    pltpu.sync_copy(
                        spacc.at[pl.ds(15 * 528, 416)],
                        agg_user.at[pl.ds(ulo + 15 * 528, 416)])

            @pl.when(rid == 5)
            def _():
                pltpu.sync_copy(spacc.at[pl.ds(s * 520, 520)],
                                agg_user.at[pl.ds(ulo + s * 520, 520)])

        do_pass(x_item, src_i2u, dst_i2u, w_i2u, ulo, ulo + usz, flush_user)


@jax.jit
def _sc_aggregate(x_user, x_item, src_u2i, dst_u2i, w_u2i, src_i2u,
                  dst_i2u, w_i2u):
    mesh = plsc.VectorSubcoreMesh(core_axis_name="c", subcore_axis_name="s",
                                  num_cores=NC, num_subcores=NS)
    f = pl.kernel(
        _sc_body,
        out_type=[jax.ShapeDtypeStruct((N_ITEM, D), jnp.float32),
                  jax.ShapeDtypeStruct((N_USER, D), jnp.float32)],
        mesh=mesh,
        scratch_types=[
            pltpu.VMEM((BLK_E,), jnp.int32),
            pltpu.VMEM((BLK_E,), jnp.int32),
            pltpu.VMEM((BLK_E,), jnp.float32),
            pltpu.VMEM((CCAP,), jnp.int32),
            pltpu.VMEM((CCAP,), jnp.int32),
            pltpu.VMEM((CCAP,), jnp.float32),
            pltpu.VMEM((NSLOT, CHUNK), jnp.int32),
            pltpu.VMEM((NSLOT, CHUNK), jnp.int32),
            pltpu.VMEM((NSLOT, CHUNK, D), jnp.float32),
            pltpu.VMEM_SHARED((SP_ROWS, D), jnp.float32),
            pltpu.SemaphoreType.DMA((NSLOT,)),
            pltpu.SemaphoreType.DMA,
        ],
        compiler_params=pltpu.CompilerParams(needs_layout_passes=False),
    )
    return f(x_user, x_item, src_u2i, dst_u2i, w_u2i, src_i2u, dst_i2u,
             w_i2u)


def _dense_body(agg_ref, x_ref, wm_ref, ws_ref, b_ref, o_ref):
    o_ref[...] = (
        jnp.dot(agg_ref[...], wm_ref[...], preferred_element_type=jnp.float32)
        + jnp.dot(x_ref[...], ws_ref[...], preferred_element_type=jnp.float32)
        + b_ref[...])


def _dense(agg, x, Wm, Ws, b, blk):
    n = agg.shape[0]
    return pl.pallas_call(
        _dense_body,
        grid=(n // blk,),
        in_specs=[
            pl.BlockSpec((blk, D), lambda i: (i, 0)),
            pl.BlockSpec((blk, D), lambda i: (i, 0)),
            pl.BlockSpec((D, D), lambda i: (0, 0)),
            pl.BlockSpec((D, D), lambda i: (0, 0)),
            pl.BlockSpec((1, D), lambda i: (0, 0)),
        ],
        out_specs=pl.BlockSpec((blk, D), lambda i: (i, 0)),
        out_shape=jax.ShapeDtypeStruct((n, D), jnp.float32),
    )(agg, x, Wm, Ws, b.reshape(1, D))


def kernel(x_user, x_item, src_u2i, dst_u2i, edge_weight_u2i, src_i2u,
           dst_i2u, edge_weight_i2u, W_msg_u2i, W_self_u2i, b_u2i,
           W_msg_i2u, W_self_i2u, b_i2u):
    src_u2i = src_u2i.astype(jnp.int32)
    dst_u2i = dst_u2i.astype(jnp.int32)
    src_i2u = src_i2u.astype(jnp.int32)
    dst_i2u = dst_i2u.astype(jnp.int32)
    agg_item, agg_user = _sc_aggregate(
        x_user, x_item, src_u2i, dst_u2i, edge_weight_u2i, src_i2u,
        dst_i2u, edge_weight_i2u)
    out_item = _dense(agg_item, x_item, W_msg_u2i, W_self_u2i, b_u2i, 2000)
    out_user = _dense(agg_user, x_user, W_msg_i2u, W_self_i2u, b_i2u, 2000)
    return out_user, out_item


# speedup vs baseline: 2.1711x; 2.1711x over previous
import jax
import jax.numpy as jnp
from jax import lax
from jax.experimental import pallas as pl
from jax.experimental.pallas import tpu as pltpu
from jax.experimental.pallas import tpu_sc as plsc

N_USER = 50000
N_ITEM = 10000
D = 128
E = 320000

NC = 2
NS = 16
EP = E // NS
BLK_E = 2000
NBLK = EP // BLK_E
CHUNK = 128
CCAP = 2208
CTRASH = CCAP - 16
SP_ROWS = 8448
ZPT = 528
TRASH_ROW = 8336

U_SZ_MAIN = 8336
U_SZ_LAST = 8320
IR = 5000


def _sc_body(x_user, x_item, src_u2i, dst_u2i, w_u2i, src_i2u, dst_i2u,
             w_i2u, agg_item, agg_user, bdst, bsrc, bw, cdst, csrc, cw,
             didx, sidx, rows, spacc, gsem, ssem):
    c = lax.axis_index("c")
    s = lax.axis_index("s")

    def do_pass(table, src_hbm, dst_hbm, w_hbm, lo, hi, flush):
        def stage_fire(i, nb):
            base = i * CHUNK
            for j in range(CHUNK // 16):
                didx[nb, pl.ds(j * 16, 16)] = cdst[pl.ds(base + j * 16, 16)]
                sidx[nb, pl.ds(j * 16, 16)] = csrc[pl.ds(base + j * 16, 16)]
            pltpu.async_copy(table.at[sidx.at[nb]], rows.at[nb],
                             gsem.at[nb])

        def scale_scatter(i, b):
            base = i * CHUNK

            def scale(e, _):
                w = plsc.load_gather(cw,
                                     [jnp.full((16,), base + e, jnp.int32)])
                for j in range(8):
                    sl = pl.ds(j * 16, 16)
                    rows[b, e, sl] = rows[b, e, sl] * w
                return 0
            lax.fori_loop(0, CHUNK, scale, 0, unroll=4)
            pltpu.sync_copy(rows.at[b], spacc.at[didx.at[b]], add=True)

        def run_chunks(n_full):
            @pl.when(n_full > 0)
            def _():
                stage_fire(0, 0)

            def ch_body(i, _):
                b = jnp.bitwise_and(i, 1)
                pltpu.make_async_copy(table.at[sidx.at[b]], rows.at[b],
                                      gsem.at[b]).wait()

                @pl.when(i + 1 < n_full)
                def _():
                    stage_fire(i + 1, 1 - b)
                scale_scatter(i, b)
                return 0
            lax.fori_loop(0, n_full, ch_body, 0)

        def _zrow(r, _):
            for j in range(8):
                rows[0, r, pl.ds(j * 16, 16)] = jnp.zeros((16,),
                                                          jnp.float32)
            return 0
        lax.fori_loop(0, CHUNK, _zrow, 0)
        zbase = s * ZPT
        for k in range(4):
            pltpu.sync_copy(rows.at[0],
                            spacc.at[pl.ds(zbase + k * 128, 128)])
        pltpu.sync_copy(rows.at[0].at[pl.ds(0, 16)],
                        spacc.at[pl.ds(zbase + 512, 16)])
        plsc.subcore_barrier()
        ep0 = s * EP
        pltpu.async_copy(dst_hbm.at[pl.ds(ep0, BLK_E)], bdst, ssem)
        pltpu.async_copy(src_hbm.at[pl.ds(ep0, BLK_E)], bsrc, ssem)
        pltpu.async_copy(w_hbm.at[pl.ds(ep0, BLK_E)], bw, ssem)

        def vec_body(v, cnt):
            sl = pl.ds(v * 16, 16)
            d = bdst[sl]
            m = (d >= lo) & (d < hi)
            ms = m.astype(jnp.int32)
            pref = lax.cumsum(ms)
            pos = jnp.where(m, cnt + pref - ms,
                            CTRASH + lax.iota(jnp.int32, 16))
            plsc.store_scatter(cdst, [pos], d - lo)
            plsc.store_scatter(csrc, [pos], bsrc[sl])
            plsc.store_scatter(cw, [pos], bw[sl])
            return cnt + plsc.all_reduce_population_count(m)

        def blk_body(blk, cnt):
            off = s * EP + blk * BLK_E
            pltpu.make_async_copy(dst_hbm.at[pl.ds(off, BLK_E)], bdst,
                                  ssem).wait()
            pltpu.make_async_copy(src_hbm.at[pl.ds(off, BLK_E)], bsrc,
                                  ssem).wait()
            pltpu.make_async_copy(w_hbm.at[pl.ds(off, BLK_E)], bw,
                                  ssem).wait()
            cnt = lax.fori_loop(0, BLK_E // 16, vec_body, cnt, unroll=2)

            @pl.when(blk + 1 < NBLK)
            def _():
                off2 = off + BLK_E
                pltpu.async_copy(dst_hbm.at[pl.ds(off2, BLK_E)], bdst, ssem)
                pltpu.async_copy(src_hbm.at[pl.ds(off2, BLK_E)], bsrc, ssem)
                pltpu.async_copy(w_hbm.at[pl.ds(off2, BLK_E)], bw, ssem)
            n_full = cnt[0] // CHUNK
            run_chunks(n_full)
            rbase = n_full * CHUNK
            for j in range(CHUNK // 16):
                so = pl.ds(rbase + j * 16, 16)
                do = pl.ds(j * 16, 16)
                dv, sv, wv = cdst[so], csrc[so], cw[so]
                cdst[do] = dv
                csrc[do] = sv
                cw[do] = wv
            return cnt - rbase

        cntv = lax.fori_loop(0, NBLK, blk_body,
                             jnp.zeros((16,), jnp.int32))
        cnt = cntv[0]
        trash = jnp.full((16,), TRASH_ROW + s, jnp.int32)
        zf = jnp.zeros((16,), jnp.float32)
        for i in range(CHUNK // 16):
            off = pl.ds(cnt + i * 16, 16)
            cdst[off] = trash
            csrc[off] = (s * CHUNK + i * 16) + lax.iota(jnp.int32, 16)
            cw[off] = zf
        run_chunks((cnt + CHUNK - 1) // CHUNK)
        plsc.subcore_barrier()
        flush()
        plsc.subcore_barrier()

    ilo = c * IR

    def flush_item():
        @pl.when(s < 15)
        def _():
            pltpu.sync_copy(spacc.at[pl.ds(s * 312, 312)],
                            agg_item.at[pl.ds(ilo + s * 312, 312)])

        @pl.when(s == 15)
        def _():
            pltpu.sync_copy(spacc.at[pl.ds(15 * 312, 320)],
                            agg_item.at[pl.ds(ilo + 15 * 312, 320)])

    do_pass(x_user, src_u2i, dst_u2i, w_u2i, ilo, ilo + IR, flush_item)

    for p in range(3):
        rid = c * 3 + p
        ulo = rid * U_SZ_MAIN
        usz = jnp.where(rid == 5, U_SZ_LAST, U_SZ_MAIN).astype(jnp.int32)

        def flush_user(ulo=ulo, rid=rid):
            @pl.when(rid < 5)
            def _():
                @pl.when(s < 15)
                def _():
                    pltpu.sync_copy(spacc.at[pl.ds(s * 528, 528)],
                                    agg_user.at[pl.ds(ulo + s * 528, 528)])

                @pl.when(s == 15)
                def _():
                    pltpu.sync_copy(
                        spacc.at[pl.ds(15 * 528, 416)],
                        agg_user.at[pl.ds(ulo + 15 * 528, 416)])

            @pl.when(rid == 5)
            def _():
                pltpu.sync_copy(spacc.at[pl.ds(s * 520, 520)],
                                agg_user.at[pl.ds(ulo + s * 520, 520)])

        do_pass(x_item, src_i2u, dst_i2u, w_i2u, ulo, ulo + usz, flush_user)


@jax.jit
def _sc_aggregate(x_user, x_item, src_u2i, dst_u2i, w_u2i, src_i2u,
                  dst_i2u, w_i2u):
    mesh = plsc.VectorSubcoreMesh(core_axis_name="c", subcore_axis_name="s",
                                  num_cores=NC, num_subcores=NS)
    f = pl.kernel(
        _sc_body,
        out_type=[jax.ShapeDtypeStruct((N_ITEM, D), jnp.float32),
                  jax.ShapeDtypeStruct((N_USER, D), jnp.float32)],
        mesh=mesh,
        scratch_types=[
            pltpu.VMEM((BLK_E,), jnp.int32),
            pltpu.VMEM((BLK_E,), jnp.int32),
            pltpu.VMEM((BLK_E,), jnp.float32),
            pltpu.VMEM((CCAP,), jnp.int32),
            pltpu.VMEM((CCAP,), jnp.int32),
            pltpu.VMEM((CCAP,), jnp.float32),
            pltpu.VMEM((2, CHUNK), jnp.int32),
            pltpu.VMEM((2, CHUNK), jnp.int32),
            pltpu.VMEM((2, CHUNK, D), jnp.float32),
            pltpu.VMEM_SHARED((SP_ROWS, D), jnp.float32),
            pltpu.SemaphoreType.DMA((2,)),
            pltpu.SemaphoreType.DMA,
        ],
        compiler_params=pltpu.CompilerParams(needs_layout_passes=False),
    )
    return f(x_user, x_item, src_u2i, dst_u2i, w_u2i, src_i2u, dst_i2u,
             w_i2u)


def _dense_body(agg_ref, x_ref, wm_ref, ws_ref, b_ref, o_ref):
    o_ref[...] = (
        jnp.dot(agg_ref[...], wm_ref[...], preferred_element_type=jnp.float32)
        + jnp.dot(x_ref[...], ws_ref[...], preferred_element_type=jnp.float32)
        + b_ref[...])


def _dense(agg, x, Wm, Ws, b, blk):
    n = agg.shape[0]
    return pl.pallas_call(
        _dense_body,
        grid=(n // blk,),
        in_specs=[
            pl.BlockSpec((blk, D), lambda i: (i, 0)),
            pl.BlockSpec((blk, D), lambda i: (i, 0)),
            pl.BlockSpec((D, D), lambda i: (0, 0)),
            pl.BlockSpec((D, D), lambda i: (0, 0)),
            pl.BlockSpec((1, D), lambda i: (0, 0)),
        ],
        out_specs=pl.BlockSpec((blk, D), lambda i: (i, 0)),
        out_shape=jax.ShapeDtypeStruct((n, D), jnp.float32),
    )(agg, x, Wm, Ws, b.reshape(1, D))


def kernel(x_user, x_item, src_u2i, dst_u2i, edge_weight_u2i, src_i2u,
           dst_i2u, edge_weight_i2u, W_msg_u2i, W_self_u2i, b_u2i,
           W_msg_i2u, W_self_i2u, b_i2u):
    src_u2i = src_u2i.astype(jnp.int32)
    dst_u2i = dst_u2i.astype(jnp.int32)
    src_i2u = src_i2u.astype(jnp.int32)
    dst_i2u = dst_i2u.astype(jnp.int32)
    agg_item, agg_user = _sc_aggregate(
        x_user, x_item, src_u2i, dst_u2i, edge_weight_u2i, src_i2u,
        dst_i2u, edge_weight_i2u)
    out_item = _dense(agg_item, x_item, W_msg_u2i, W_self_u2i, b_u2i, 2000)
    out_user = _dense(agg_user, x_user, W_msg_i2u, W_self_i2u, b_i2u, 2000)
    return out_user, out_item
